# Initial kernel scaffold; baseline (speedup 1.0000x reference)
import jax
import jax.numpy as jnp
from jax.experimental import pallas as pl


def kernel(item_id, batch_size, item_bias):
    raise NotImplementedError("write your pallas kernel here")


# speedup vs baseline: 1.0181x; 1.0181x over previous
import functools

import jax
import jax.numpy as jnp
from jax import lax
from jax.experimental import pallas as pl
from jax.experimental.pallas import tpu as pltpu
from jax.experimental.pallas import tpu_sc as plsc

_NUM_CORES = 2
_NUM_SUBCORES = 16
_NUM_WORKERS = _NUM_CORES * _NUM_SUBCORES


def _gather_body(n_per_worker, idx_hbm, table_hbm, out_hbm, idx_v, vals_v, sem):
    wid = lax.axis_index("s") * _NUM_CORES + lax.axis_index("c")
    base = wid * n_per_worker
    pltpu.sync_copy(idx_hbm.at[pl.ds(base, n_per_worker)], idx_v)
    pltpu.async_copy(table_hbm.at[idx_v], vals_v, sem).wait()
    pltpu.sync_copy(vals_v, out_hbm.at[pl.ds(base, n_per_worker)])


def kernel(item_id, batch_size, item_bias):
    b, n = item_id.shape
    total = b * n
    n_per_worker = total // _NUM_WORKERS
    idx = item_id.reshape(total).astype(jnp.int32)
    table = item_bias.reshape(-1)

    mesh = plsc.VectorSubcoreMesh(core_axis_name="c", subcore_axis_name="s")
    out = pl.kernel(
        functools.partial(_gather_body, n_per_worker),
        out_type=jax.ShapeDtypeStruct((total,), jnp.float32),
        mesh=mesh,
        scratch_types=[
            pltpu.VMEM((n_per_worker,), jnp.int32),
            pltpu.VMEM((n_per_worker,), jnp.float32),
            pltpu.SemaphoreType.DMA,
        ],
    )(idx, table)
    return out.reshape(b, n)


# speedup vs baseline: 1.3745x; 1.3500x over previous
import functools

import jax
import jax.numpy as jnp
from jax import lax
from jax.experimental import pallas as pl
from jax.experimental.pallas import tpu as pltpu
from jax.experimental.pallas import tpu_sc as plsc

_NUM_CORES = 2
_NUM_SUBCORES = 16
_NUM_WORKERS = _NUM_CORES * _NUM_SUBCORES


def _gather_body(rows_per_worker,
                 idx0_hbm, idx1_hbm, table_hbm,
                 out0_hbm, out1_hbm,
                 idx_v, vals_v, sem):
    wid = lax.axis_index("s") * _NUM_CORES + lax.axis_index("c")
    base = wid * rows_per_worker
    for idx_hbm, out_hbm in ((idx0_hbm, out0_hbm), (idx1_hbm, out1_hbm)):
        pltpu.sync_copy(idx_hbm.at[pl.ds(base, rows_per_worker)], idx_v)
        pltpu.async_copy(table_hbm.at[idx_v], vals_v, sem).wait()
        pltpu.sync_copy(vals_v, out_hbm.at[pl.ds(base, rows_per_worker)])


def kernel(item_id, batch_size, item_bias):
    b, n = item_id.shape
    rows_per_worker = b // _NUM_WORKERS
    table = item_bias.reshape(-1)
    idx0 = item_id[:, 0]
    idx1 = item_id[:, 1]

    mesh = plsc.VectorSubcoreMesh(core_axis_name="c", subcore_axis_name="s")
    out0, out1 = pl.kernel(
        functools.partial(_gather_body, rows_per_worker),
        out_type=(
            jax.ShapeDtypeStruct((b,), jnp.float32),
            jax.ShapeDtypeStruct((b,), jnp.float32),
        ),
        mesh=mesh,
        scratch_types=[
            pltpu.VMEM((rows_per_worker,), jnp.int32),
            pltpu.VMEM((rows_per_worker,), jnp.float32),
            pltpu.SemaphoreType.DMA,
        ],
    )(idx0, idx1, table)
    return jnp.stack([out0, out1], axis=-1)


# speedup vs baseline: 1.3926x; 1.0132x over previous
import functools

import jax
import jax.numpy as jnp
from jax import lax
from jax.experimental import pallas as pl
from jax.experimental.pallas import tpu as pltpu
from jax.experimental.pallas import tpu_sc as plsc

_NUM_CORES = 2
_NUM_SUBCORES = 16
_NUM_WORKERS = _NUM_CORES * _NUM_SUBCORES


def _gather_body(rows_per_worker,
                 idx0_hbm, idx1_hbm, table_hbm,
                 out0_hbm, out1_hbm,
                 idx0_v, idx1_v, vals0_v, vals1_v, sem0, sem1):
    wid = lax.axis_index("s") * _NUM_CORES + lax.axis_index("c")
    base = wid * rows_per_worker
    sl = pl.ds(base, rows_per_worker)
    pltpu.sync_copy(idx0_hbm.at[sl], idx0_v)
    pltpu.sync_copy(idx1_hbm.at[sl], idx1_v)
    g0 = pltpu.async_copy(table_hbm.at[idx0_v], vals0_v, sem0)
    g1 = pltpu.async_copy(table_hbm.at[idx1_v], vals1_v, sem1)
    g0.wait()
    g1.wait()
    pltpu.sync_copy(vals0_v, out0_hbm.at[sl])
    pltpu.sync_copy(vals1_v, out1_hbm.at[sl])


def kernel(item_id, batch_size, item_bias):
    b, n = item_id.shape
    rows_per_worker = b // _NUM_WORKERS
    table = item_bias.reshape(-1)
    idx0 = item_id[:, 0]
    idx1 = item_id[:, 1]

    mesh = plsc.VectorSubcoreMesh(core_axis_name="c", subcore_axis_name="s")
    out0, out1 = pl.kernel(
        functools.partial(_gather_body, rows_per_worker),
        out_type=(
            jax.ShapeDtypeStruct((b,), jnp.float32),
            jax.ShapeDtypeStruct((b,), jnp.float32),
        ),
        mesh=mesh,
        scratch_types=[
            pltpu.VMEM((rows_per_worker,), jnp.int32),
            pltpu.VMEM((rows_per_worker,), jnp.int32),
            pltpu.VMEM((rows_per_worker,), jnp.float32),
            pltpu.VMEM((rows_per_worker,), jnp.float32),
            pltpu.SemaphoreType.DMA,
            pltpu.SemaphoreType.DMA,
        ],
    )(idx0, idx1, table)
    return jnp.stack([out0, out1], axis=-1)
